# baseline (speedup 1.0000x reference)
import functools

import jax
import jax.numpy as jnp
from jax import lax
from jax.experimental import pallas as pl
from jax.experimental.pallas import tpu as pltpu
from jax.experimental.pallas import tpu_sc as plsc

D = 64
B = 16384
K = 20

NC, NS, L = 2, 16, 16
NW = NC * NS
PER_W = B // NW
CHUNK = 32
NSTEP = PER_W // CHUNK
NNEG = CHUNK * K
NGATH = NNEG // 128


def _sc_scores_body(cen_hbm, ctx_hbm, neg_hbm, win_hbm, wout_hbm,
                    pos_out, neg_out,
                    cidx, xidx, nidx, cp_i, xp_i, np_i,
                    crows, prows, nrows, psc, nsc, sem):
    wid = lax.axis_index("s") * NC + lax.axis_index("c")

    def step(c, _):
        base = wid * PER_W + c * CHUNK
        pltpu.sync_copy(cen_hbm.at[pl.ds(base, CHUNK)], cidx)
        pltpu.sync_copy(ctx_hbm.at[pl.ds(base, CHUNK)], xidx)
        pltpu.sync_copy(neg_hbm.at[pl.ds(base * K, NNEG)], nidx)
        for j in range(CHUNK // L):
            cp_i[pl.ds(j * L, L)] = lax.shift_right_logical(
                cidx[pl.ds(j * L, L)], 1)
            xp_i[pl.ds(j * L, L)] = lax.shift_right_logical(
                xidx[pl.ds(j * L, L)], 1)
        for j in range(NNEG // L):
            np_i[pl.ds(j * L, L)] = lax.shift_right_logical(
                nidx[pl.ds(j * L, L)], 1)
        cps = [pltpu.async_copy(win_hbm.at[cp_i], crows, sem),
               pltpu.async_copy(wout_hbm.at[xp_i], prows, sem)]
        for j in range(NGATH):
            cps.append(pltpu.async_copy(wout_hbm.at[np_i.at[pl.ds(j * 128, 128)]],
                                        nrows.at[pl.ds(j * 128, 128)], sem))
        for cp in cps:
            cp.wait()

        for g in range(CHUNK // L):
            bvec = g * L + lax.iota(jnp.int32, L)
            choff = (cidx[pl.ds(g * L, L)] & 1) * D
            xhoff = (xidx[pl.ds(g * L, L)] & 1) * D
            nrow = [bvec * K + k for k in range(K)]
            nhoff = [(plsc.load_gather(nidx, [nrow[k]]) & 1) * D
                     for k in range(K)]

            def dstep(dd, accs, bvec=bvec, choff=choff, xhoff=xhoff,
                      nrow=nrow, nhoff=nhoff):
                dsplat = jnp.full((L,), dd, jnp.int32)
                cen_d = plsc.load_gather(crows, [bvec, choff + dsplat])
                pos_d = plsc.load_gather(prows, [bvec, xhoff + dsplat])
                out = [accs[0] + cen_d * pos_d]
                for k in range(K):
                    neg_d = plsc.load_gather(nrows, [nrow[k], nhoff[k] + dsplat])
                    out.append(accs[k + 1] + neg_d * cen_d)
                return tuple(out)

            accs = lax.fori_loop(
                0, D, dstep, (jnp.zeros((L,), jnp.float32),) * (K + 1))
            psc[pl.ds(c * CHUNK + g * L, L)] = accs[0]
            for k in range(K):
                plsc.store_scatter(nsc, [c * NNEG + nrow[k]], accs[k + 1])
        return 0

    lax.fori_loop(0, NSTEP, step, 0)
    pltpu.sync_copy(psc, pos_out.at[pl.ds(wid * PER_W, PER_W)])
    pltpu.sync_copy(nsc, neg_out.at[pl.ds(wid * PER_W * K, PER_W * K)])


@jax.jit
def _sc_scores(cen, ctx, neg1d, w_in2, w_out2):
    f = pl.kernel(
        _sc_scores_body,
        out_type=(jax.ShapeDtypeStruct((B,), jnp.float32),
                  jax.ShapeDtypeStruct((B * K,), jnp.float32)),
        mesh=plsc.VectorSubcoreMesh(core_axis_name="c", subcore_axis_name="s"),
        compiler_params=pltpu.CompilerParams(needs_layout_passes=False),
        scratch_types=[
            pltpu.VMEM((CHUNK,), jnp.int32),
            pltpu.VMEM((CHUNK,), jnp.int32),
            pltpu.VMEM((NNEG,), jnp.int32),
            pltpu.VMEM((CHUNK,), jnp.int32),
            pltpu.VMEM((CHUNK,), jnp.int32),
            pltpu.VMEM((NNEG,), jnp.int32),
            pltpu.VMEM((CHUNK, 2 * D), jnp.float32),
            pltpu.VMEM((CHUNK, 2 * D), jnp.float32),
            pltpu.VMEM((NNEG, 2 * D), jnp.float32),
            pltpu.VMEM((PER_W,), jnp.float32),
            pltpu.VMEM((PER_W * K,), jnp.float32),
            pltpu.SemaphoreType.DMA,
        ],
    )
    return f(cen, ctx, neg1d, w_in2, w_out2)


def _tc_loss_body(pos_ref, neg_ref, out_ref):
    pls = jax.nn.log_sigmoid(pos_ref[...])
    nls = jax.nn.log_sigmoid(-neg_ref[...])
    out_ref[0, 0] = -(jnp.sum(pls) + jnp.sum(nls)) / B


def _tc_loss(pos2d, neg2d):
    return pl.pallas_call(
        _tc_loss_body,
        out_shape=jax.ShapeDtypeStruct((1, 1), jnp.float32),
        out_specs=pl.BlockSpec(memory_space=pltpu.SMEM),
    )(pos2d, neg2d)


def kernel(center, context, negatives, W_in, W_out):
    cen = center.astype(jnp.int32)
    ctx = context.astype(jnp.int32)
    neg = negatives.astype(jnp.int32).reshape(B * K)
    w_in2 = W_in.reshape(W_in.shape[0] // 2, 2 * D)
    w_out2 = W_out.reshape(W_out.shape[0] // 2, 2 * D)
    pos_s, neg_s = _sc_scores(cen, ctx, neg, w_in2, w_out2)
    loss = _tc_loss(pos_s.reshape(B // 128, 128),
                    neg_s.reshape(B * K // 128, 128))
    return loss[0, 0]


# speedup vs baseline: 1.2280x; 1.2280x over previous
import jax
import jax.numpy as jnp
from jax import lax
from jax.experimental import pallas as pl
from jax.experimental.pallas import tpu as pltpu
from jax.experimental.pallas import tpu_sc as plsc

D = 64
B = 16384
K = 20

NC, NS, L = 2, 16, 16
NW = NC * NS
PER_W = B // NW
CHUNK = 64
NSTEP = PER_W // CHUNK
NNEG = CHUNK * K
NGATH = NNEG // 128


def _sc_scores_body(cen_hbm, ctx_hbm, neg_hbm, win_hbm, wout_hbm,
                    pos_out, neg_out,
                    cidx, xidx, nidx, crows, prows, nrows, psc, nsc, sem):
    wid = lax.axis_index("s") * NC + lax.axis_index("c")

    def step(c, _):
        base = wid * PER_W + c * CHUNK
        cps = [pltpu.async_copy(cen_hbm.at[pl.ds(base, CHUNK)], cidx, sem),
               pltpu.async_copy(ctx_hbm.at[pl.ds(base, CHUNK)], xidx, sem)]
        for k in range(K):
            cps.append(pltpu.async_copy(neg_hbm.at[pl.ds(k * B + base, CHUNK)],
                                        nidx.at[pl.ds(k * CHUNK, CHUNK)], sem))
        for cp in cps:
            cp.wait()
        cps = [pltpu.async_copy(win_hbm.at[cidx], crows, sem),
               pltpu.async_copy(wout_hbm.at[xidx], prows, sem)]
        for j in range(NGATH):
            cps.append(pltpu.async_copy(wout_hbm.at[nidx.at[pl.ds(j * 128, 128)]],
                                        nrows.at[pl.ds(j * 128, 128)], sem))
        for cp in cps:
            cp.wait()

        lane15 = lax.iota(jnp.int32, L) == (L - 1)

        def brow(b, _):
            cvs = [crows[b, pl.ds(j * L, L)] for j in range(D // L)]
            pvs = [prows[b, pl.ds(j * L, L)] for j in range(D // L)]
            s = plsc.cumsum(sum(cv * pv for cv, pv in zip(cvs, pvs)))
            plsc.store_scatter(psc, [jnp.full((L,), c * CHUNK + b, jnp.int32)],
                               s, mask=lane15)
            for k in range(K):
                nvs = [nrows[k * CHUNK + b, pl.ds(j * L, L)]
                       for j in range(D // L)]
                t = plsc.cumsum(sum(cv * nv for cv, nv in zip(cvs, nvs)))
                plsc.store_scatter(
                    nsc, [jnp.full((L,), k * PER_W + c * CHUNK + b, jnp.int32)],
                    t, mask=lane15)
            return 0

        lax.fori_loop(0, CHUNK, brow, 0)
        return 0

    lax.fori_loop(0, NSTEP, step, 0)
    pltpu.sync_copy(psc, pos_out.at[pl.ds(wid * PER_W, PER_W)])
    for k in range(K):
        pltpu.sync_copy(nsc.at[pl.ds(k * PER_W, PER_W)],
                        neg_out.at[pl.ds(k * B + wid * PER_W, PER_W)])


@jax.jit
def _sc_scores(cen, ctx, neg1d, w_in, w_out):
    f = pl.kernel(
        _sc_scores_body,
        out_type=(jax.ShapeDtypeStruct((B,), jnp.float32),
                  jax.ShapeDtypeStruct((B * K,), jnp.float32)),
        mesh=plsc.VectorSubcoreMesh(core_axis_name="c", subcore_axis_name="s"),
        compiler_params=pltpu.CompilerParams(needs_layout_passes=False,
                                             use_tc_tiling_on_sc=False),
        scratch_types=[
            pltpu.VMEM((CHUNK,), jnp.int32),
            pltpu.VMEM((CHUNK,), jnp.int32),
            pltpu.VMEM((NNEG,), jnp.int32),
            pltpu.VMEM((CHUNK, D), jnp.float32),
            pltpu.VMEM((CHUNK, D), jnp.float32),
            pltpu.VMEM((NNEG, D), jnp.float32),
            pltpu.VMEM((PER_W,), jnp.float32),
            pltpu.VMEM((K * PER_W,), jnp.float32),
            pltpu.SemaphoreType.DMA,
        ],
    )
    return f(cen, ctx, neg1d, w_in, w_out)


def _tc_loss_body(pos_ref, neg_ref, out_ref):
    pls = jax.nn.log_sigmoid(pos_ref[...])
    nls = jax.nn.log_sigmoid(-neg_ref[...])
    out_ref[0, 0] = -(jnp.sum(pls) + jnp.sum(nls)) / B


def _tc_loss(pos2d, neg2d):
    return pl.pallas_call(
        _tc_loss_body,
        out_shape=jax.ShapeDtypeStruct((1, 1), jnp.float32),
        out_specs=pl.BlockSpec(memory_space=pltpu.SMEM),
    )(pos2d, neg2d)


def kernel(center, context, negatives, W_in, W_out):
    cen = center.astype(jnp.int32)
    ctx = context.astype(jnp.int32)
    neg = negatives.astype(jnp.int32).T.reshape(B * K)
    pos_s, neg_s = _sc_scores(cen, ctx, neg, W_in, W_out)
    loss = _tc_loss(pos_s.reshape(B // 128, 128),
                    neg_s.reshape(B * K // 128, 128))
    return loss[0, 0]
